# baseline (device time: 235946 ns/iter reference)
import jax
import jax.numpy as jnp
from jax import lax
from jax.experimental import pallas as pl
from jax.experimental.pallas import tpu as pltpu

B = 32
H = 16
D = 128
BS = 32
NB = 256
P_LOCAL = 256
KB_PAGES = 8
KB_TOK = KB_PAGES * BS
N_KB = P_LOCAL // KB_PAGES
NEG = -1e30
SCALE = D ** -0.5


def _attn_body(q_ref, k_ref, v_ref, bt_ref, lens_ref,
               acc_ref, m_ref, l_ref, bias_ref):
    kb = pl.program_id(0)
    my_y = lax.axis_index("y")

    @pl.when(kb == 0)
    def _init():
        m_ref[...] = jnp.full((H, B, 1), NEG, jnp.float32)
        l_ref[...] = jnp.zeros((H, B, 1), jnp.float32)
        acc_ref[...] = jnp.zeros((H, B, D), jnp.float32)
        bt = bt_ref[...]
        lens = lens_ref[...]
        slot = lax.broadcasted_iota(jnp.int32, (1, 1, NB), 2)
        valid = slot < lens[None, :, :]
        base = my_y * P_LOCAL
        CH = 32
        for c in range(P_LOCAL // CH):
            pages = base + c * CH + lax.broadcasted_iota(
                jnp.int32, (CH, 1, 1), 0)
            eq = bt[None, :, :] == pages
            cnt = jnp.sum(jnp.where(eq & valid, 1.0, 0.0), axis=2)
            bias_ref[c * CH:(c + 1) * CH, :] = jnp.where(
                cnt > 0.5, jnp.log(cnt), NEG)

    bias_blk = bias_ref[pl.ds(kb * KB_PAGES, KB_PAGES), :]
    rows = lax.broadcasted_iota(jnp.int32, (KB_PAGES, KB_TOK), 0)
    cols = lax.broadcasted_iota(jnp.int32, (KB_PAGES, KB_TOK), 1)
    expand = jnp.where(cols // BS == rows, 1.0, 0.0)
    bias_tok = lax.dot_general(
        bias_blk, expand, (((0,), (0,)), ((), ())),
        preferred_element_type=jnp.float32)

    for h in range(H):
        qh = q_ref[h].astype(jnp.bfloat16)
        kh = k_ref[:, h, :].astype(jnp.bfloat16)
        vh = v_ref[:, h, :].astype(jnp.bfloat16)
        s = lax.dot_general(
            qh, kh, (((1,), (1,)), ((), ())),
            preferred_element_type=jnp.float32)
        s = s * SCALE + bias_tok
        m_old = m_ref[h]
        m_new = jnp.maximum(m_old, jnp.max(s, axis=1, keepdims=True))
        p = jnp.exp(s - m_new)
        corr = jnp.exp(m_old - m_new)
        m_ref[h] = m_new
        l_ref[h] = l_ref[h] * corr + jnp.sum(p, axis=1, keepdims=True)
        pv = lax.dot_general(
            p.astype(jnp.bfloat16), vh, (((1,), (0,)), ((), ())),
            preferred_element_type=jnp.float32)
        acc_ref[h] = acc_ref[h] * corr + pv


def _partial(q, k, v, bt, lens2):
    return pl.pallas_call(
        _attn_body,
        grid=(N_KB,),
        in_specs=[
            pl.BlockSpec((H, B, D), lambda kb: (0, 0, 0)),
            pl.BlockSpec((KB_TOK, H, D), lambda kb: (kb, 0, 0)),
            pl.BlockSpec((KB_TOK, H, D), lambda kb: (kb, 0, 0)),
            pl.BlockSpec((B, NB), lambda kb: (0, 0)),
            pl.BlockSpec((B, 1), lambda kb: (0, 0)),
        ],
        out_specs=[
            pl.BlockSpec((H, B, D), lambda kb: (0, 0, 0)),
            pl.BlockSpec((H, B, 1), lambda kb: (0, 0, 0)),
            pl.BlockSpec((H, B, 1), lambda kb: (0, 0, 0)),
        ],
        out_shape=[
            jax.ShapeDtypeStruct((H, B, D), jnp.float32),
            jax.ShapeDtypeStruct((H, B, 1), jnp.float32),
            jax.ShapeDtypeStruct((H, B, 1), jnp.float32),
        ],
        scratch_shapes=[pltpu.VMEM((P_LOCAL, B), jnp.float32)],
        compiler_params=pltpu.CompilerParams(
            dimension_semantics=("arbitrary",)),
    )(q, k, v, bt, lens2)


def _combine_body(acc_ref, m_ref, l_ref, out_ref,
                  r_acc, r_m, r_l, send_sems, recv_sems):
    my_x = lax.axis_index("x")
    my_y = lax.axis_index("y")
    peer = (my_x, 1 - my_y)

    barrier = pltpu.get_barrier_semaphore()
    pl.semaphore_signal(barrier, inc=1, device_id=peer,
                        device_id_type=pl.DeviceIdType.MESH)
    pl.semaphore_wait(barrier, 1)

    copies = []
    for i, (src, dst) in enumerate(
            ((acc_ref, r_acc), (m_ref, r_m), (l_ref, r_l))):
        rdma = pltpu.make_async_remote_copy(
            src_ref=src, dst_ref=dst,
            send_sem=send_sems.at[i], recv_sem=recv_sems.at[i],
            device_id=peer, device_id_type=pl.DeviceIdType.MESH)
        rdma.start()
        copies.append(rdma)
    for rdma in copies:
        rdma.wait()

    m, l, acc = m_ref[...], l_ref[...], acc_ref[...]
    mr, lr, ar = r_m[...], r_l[...], r_acc[...]
    mt = jnp.maximum(m, mr)
    a = jnp.exp(m - mt)
    b = jnp.exp(mr - mt)
    lt = a * l + b * lr
    o = (a * acc + b * ar) / lt
    for h in range(H):
        out_ref[:, 0, h, :] = o[h]


def _combine(acc, m, l):
    return pl.pallas_call(
        _combine_body,
        in_specs=[pl.BlockSpec(memory_space=pltpu.VMEM)] * 3,
        out_specs=pl.BlockSpec(memory_space=pltpu.VMEM),
        out_shape=jax.ShapeDtypeStruct((B, 1, H, D), jnp.float32),
        scratch_shapes=[
            pltpu.VMEM((H, B, D), jnp.float32),
            pltpu.VMEM((H, B, 1), jnp.float32),
            pltpu.VMEM((H, B, 1), jnp.float32),
            pltpu.SemaphoreType.DMA((3,)),
            pltpu.SemaphoreType.DMA((3,)),
        ],
        compiler_params=pltpu.CompilerParams(collective_id=0),
    )(acc, m, l)


def kernel(Q, K, V, bt, lens):
    q = jnp.transpose(Q.reshape(B, H, D), (1, 0, 2))
    k = K.reshape(P_LOCAL * BS, H, D)
    v = V.reshape(P_LOCAL * BS, H, D)
    lens2 = lens.reshape(B, 1)
    acc, m, l = _partial(q, k, v, bt, lens2)
    return _combine(acc, m, l)


# device time: 138498 ns/iter; 1.7036x vs baseline; 1.7036x over previous
import jax
import jax.numpy as jnp
from jax import lax
from jax.experimental import pallas as pl
from jax.experimental.pallas import tpu as pltpu

B = 32
H = 16
D = 128
BS = 32
NB = 256
P_LOCAL = 256
P_DEV = 128
KB_PAGES = 8
KB_TOK = KB_PAGES * BS
N_KB = P_DEV // KB_PAGES
NEG = -1e30
SCALE = D ** -0.5
MESH = pl.DeviceIdType.MESH


def _attn_body(xref, q_ref, k_ref, v_ref, bt_ref, lens_ref,
               acc_ref, m_ref, l_ref, bias_ref):
    kb = pl.program_id(0)
    my_y = lax.axis_index("y")

    @pl.when(kb == 0)
    def _init():
        m_ref[...] = jnp.full((H, B, 1), NEG, jnp.float32)
        l_ref[...] = jnp.zeros((H, B, 1), jnp.float32)
        acc_ref[...] = jnp.zeros((H, B, D), jnp.float32)
        bt = bt_ref[...]
        lens = lens_ref[...]
        slot = lax.broadcasted_iota(jnp.int32, (1, 1, NB), 2)
        valid = slot < lens[None, :, :]
        base = my_y * P_LOCAL + xref[0] * KB_PAGES
        CH = 32
        for c in range(P_DEV // CH):
            pages = base + c * CH + lax.broadcasted_iota(
                jnp.int32, (CH, 1, 1), 0)
            eq = bt[None, :, :] == pages
            cnt = jnp.sum(jnp.where(eq & valid, 1.0, 0.0), axis=2)
            bias_ref[c * CH:(c + 1) * CH, :] = jnp.where(
                cnt > 0.5, jnp.log(cnt), NEG)

    bias_blk = bias_ref[pl.ds(kb * KB_PAGES, KB_PAGES), :]
    rows = lax.broadcasted_iota(jnp.int32, (KB_PAGES, KB_TOK), 0)
    cols = lax.broadcasted_iota(jnp.int32, (KB_PAGES, KB_TOK), 1)
    expand = jnp.where(cols // BS == rows, 1.0, 0.0)
    bias_tok = lax.dot_general(
        bias_blk, expand, (((0,), (0,)), ((), ())),
        preferred_element_type=jnp.float32)

    for h in range(H):
        qh = q_ref[h].astype(jnp.bfloat16)
        kh = k_ref[:, h, :].astype(jnp.bfloat16)
        vh = v_ref[:, h, :].astype(jnp.bfloat16)
        s = lax.dot_general(
            qh, kh, (((1,), (1,)), ((), ())),
            preferred_element_type=jnp.float32)
        s = s + bias_tok
        m_old = m_ref[h]
        m_new = jnp.maximum(m_old, jnp.max(s, axis=1, keepdims=True))
        p = jnp.exp(s - m_new)
        corr = jnp.exp(m_old - m_new)
        m_ref[h] = m_new
        l_ref[h] = l_ref[h] * corr + jnp.sum(p, axis=1, keepdims=True)
        pv = lax.dot_general(
            p.astype(jnp.bfloat16), vh, (((1,), (0,)), ((), ())),
            preferred_element_type=jnp.float32)
        acc_ref[h] = acc_ref[h] * corr + pv


def _partial(xarr, q, k, v, bt, lens2):
    grid_spec = pltpu.PrefetchScalarGridSpec(
        num_scalar_prefetch=1,
        grid=(N_KB,),
        in_specs=[
            pl.BlockSpec((H, B, D), lambda kb, xr: (0, 0, 0)),
            pl.BlockSpec((KB_TOK, H, D), lambda kb, xr: (xr[0] + kb, 0, 0)),
            pl.BlockSpec((KB_TOK, H, D), lambda kb, xr: (xr[0] + kb, 0, 0)),
            pl.BlockSpec((B, NB), lambda kb, xr: (0, 0)),
            pl.BlockSpec((B, 1), lambda kb, xr: (0, 0)),
        ],
        out_specs=[
            pl.BlockSpec((H, B, D), lambda kb, xr: (0, 0, 0)),
            pl.BlockSpec((H, B, 1), lambda kb, xr: (0, 0, 0)),
            pl.BlockSpec((H, B, 1), lambda kb, xr: (0, 0, 0)),
        ],
        scratch_shapes=[pltpu.VMEM((P_DEV, B), jnp.float32)],
    )
    return pl.pallas_call(
        _attn_body,
        grid_spec=grid_spec,
        out_shape=[
            jax.ShapeDtypeStruct((H, B, D), jnp.float32),
            jax.ShapeDtypeStruct((H, B, 1), jnp.float32),
            jax.ShapeDtypeStruct((H, B, 1), jnp.float32),
        ],
        compiler_params=pltpu.CompilerParams(
            dimension_semantics=("arbitrary",)),
    )(xarr, q, k, v, bt, lens2)


def _combine_body(acc_ref, m_ref, l_ref, out_ref,
                  r_acc, r_m, r_l, s2_acc, s2_m, s2_l,
                  r2_acc, r2_m, r2_l, send_sems, recv_sems):
    my_x = lax.axis_index("x")
    my_y = lax.axis_index("y")
    y_peer = (my_x, 1 - my_y)
    x_peer = (1 - my_x, my_y)

    barrier = pltpu.get_barrier_semaphore()
    for nbr in (y_peer, x_peer):
        pl.semaphore_signal(barrier, inc=1, device_id=nbr,
                            device_id_type=MESH)
    pl.semaphore_wait(barrier, 2)

    round1 = []
    for i, (src, dst) in enumerate(
            ((acc_ref, r_acc), (m_ref, r_m), (l_ref, r_l))):
        rdma = pltpu.make_async_remote_copy(
            src_ref=src, dst_ref=dst,
            send_sem=send_sems.at[i], recv_sem=recv_sems.at[i],
            device_id=y_peer, device_id_type=MESH)
        rdma.start()
        round1.append(rdma)
    for rdma in round1:
        rdma.wait()

    m, l, acc = m_ref[...], l_ref[...], acc_ref[...]
    mr, lr, ar = r_m[...], r_l[...], r_acc[...]
    mt = jnp.maximum(m, mr)
    a = jnp.exp(m - mt)
    b = jnp.exp(mr - mt)
    s2_m[...] = mt
    s2_l[...] = a * l + b * lr
    s2_acc[...] = a * acc + b * ar

    round2 = []
    for i, (src, dst) in enumerate(
            ((s2_acc, r2_acc), (s2_m, r2_m), (s2_l, r2_l))):
        rdma = pltpu.make_async_remote_copy(
            src_ref=src, dst_ref=dst,
            send_sem=send_sems.at[3 + i], recv_sem=recv_sems.at[3 + i],
            device_id=x_peer, device_id_type=MESH)
        rdma.start()
        round2.append(rdma)
    for rdma in round2:
        rdma.wait()

    m, l, acc = s2_m[...], s2_l[...], s2_acc[...]
    mr, lr, ar = r2_m[...], r2_l[...], r2_acc[...]
    mt = jnp.maximum(m, mr)
    a = jnp.exp(m - mt)
    b = jnp.exp(mr - mt)
    lt = a * l + b * lr
    o = (a * acc + b * ar) / lt
    for h in range(H):
        out_ref[:, 0, h, :] = o[h]


def _combine(acc, m, l):
    return pl.pallas_call(
        _combine_body,
        in_specs=[pl.BlockSpec(memory_space=pltpu.VMEM)] * 3,
        out_specs=pl.BlockSpec(memory_space=pltpu.VMEM),
        out_shape=jax.ShapeDtypeStruct((B, 1, H, D), jnp.float32),
        scratch_shapes=[
            pltpu.VMEM((H, B, D), jnp.float32),
            pltpu.VMEM((H, B, 1), jnp.float32),
            pltpu.VMEM((H, B, 1), jnp.float32),
            pltpu.VMEM((H, B, D), jnp.float32),
            pltpu.VMEM((H, B, 1), jnp.float32),
            pltpu.VMEM((H, B, 1), jnp.float32),
            pltpu.VMEM((H, B, D), jnp.float32),
            pltpu.VMEM((H, B, 1), jnp.float32),
            pltpu.VMEM((H, B, 1), jnp.float32),
            pltpu.SemaphoreType.DMA((6,)),
            pltpu.SemaphoreType.DMA((6,)),
        ],
        compiler_params=pltpu.CompilerParams(collective_id=0),
    )(acc, m, l)


def kernel(Q, K, V, bt, lens):
    my_x = lax.axis_index("x")
    q = jnp.transpose(Q.reshape(B, H, D) * SCALE, (1, 0, 2))
    k = K.reshape(P_LOCAL * BS, H, D)
    v = V.reshape(P_LOCAL * BS, H, D)
    lens2 = lens.reshape(B, 1)
    xarr = jnp.full((1,), my_x * N_KB, jnp.int32)
    acc, m, l = _partial(xarr, q, k, v, bt, lens2)
    return _combine(acc, m, l)
